# baseline (speedup 1.0000x reference)
import functools

import jax
import jax.numpy as jnp
from jax import lax
from jax.experimental import pallas as pl
from jax.experimental.pallas import tpu as pltpu
from jax.experimental.pallas import tpu_sc as plsc

CH = 128
CW = 16


def _sc_gather(x, tgt):
    N, F = x.shape
    E = tgt.shape[0]
    info = plsc.get_sparse_core_info()
    NC, NS = info.num_cores, info.num_subcores
    NW = NC * NS
    assert E % CH == 0
    n_chunks = E // CH
    trips = (n_chunks + NW - 1) // NW

    mesh = plsc.VectorSubcoreMesh(core_axis_name="c", subcore_axis_name="s")

    @functools.partial(
        pl.kernel,
        out_type=jax.ShapeDtypeStruct((E, F), jnp.float32),
        mesh=mesh,
        scratch_types=[
            pltpu.VMEM((CH,), jnp.int32),
            pltpu.VMEM((CH, F), jnp.float32),
            pltpu.SemaphoreType.DMA,
        ],
    )
    def gather_k(x_hbm, tgt_hbm, out_hbm, idx_v, rows_v, sem):
        cid = lax.axis_index("c")
        sid = lax.axis_index("s")
        wid = sid * NC + cid

        def body(j, carry):
            chunk = wid + NW * j

            @pl.when(chunk < n_chunks)
            def _():
                base = chunk * CH
                pltpu.sync_copy(tgt_hbm.at[pl.ds(base, CH)], idx_v)
                pltpu.async_copy(x_hbm.at[idx_v], rows_v, sem).wait()
                pltpu.sync_copy(rows_v, out_hbm.at[pl.ds(base, CH)])

            return carry

        lax.fori_loop(0, trips, body, 0)

    return gather_k(x, tgt)


def _sc_scatter(msg, src, num_nodes):
    E, F = msg.shape
    info = plsc.get_sparse_core_info()
    NC, NS = info.num_cores, info.num_subcores
    NW = NC * NS
    NP = -(-num_nodes // (NW * 8)) * (NW * 8)
    RPT = NP // NW
    SBLK = 2048
    n_full = E // SBLK
    tail = E - n_full * SBLK
    assert tail % CH == 0

    mesh = plsc.VectorSubcoreMesh(core_axis_name="c", subcore_axis_name="s")

    @functools.partial(
        pl.kernel,
        out_type=(
            jax.ShapeDtypeStruct((NP, F), jnp.float32),
            jax.ShapeDtypeStruct((NP, CW), jnp.float32),
        ),
        mesh=mesh,
        scratch_types=[
            pltpu.VMEM((SBLK,), jnp.int32),
            pltpu.VMEM((CH, F), jnp.float32),
            pltpu.VMEM((2 * CH,), jnp.int32),
            pltpu.VMEM((2 * CH,), jnp.int32),
            pltpu.VMEM((CH,), jnp.int32),
            pltpu.VMEM((RPT, F), jnp.float32),
            pltpu.VMEM((RPT, CW), jnp.float32),
            pltpu.SemaphoreType.DMA,
        ],
        compiler_params=pltpu.CompilerParams(needs_layout_passes=False),
    )
    def scatter_k(msg_hbm, src_hbm, agg_hbm, cnt_hbm,
                  idx_v, rows_v, wl_rows, wl_eids, gbuf, acc, accc, sem):
        cid = lax.axis_index("c")
        sid = lax.axis_index("s")
        wid = sid * NC + cid
        lo = wid * RPT

        zero16 = jnp.zeros((16,), jnp.float32)
        one16 = jnp.ones((16,), jnp.float32)
        iota16 = lax.iota(jnp.int32, 16)
        zero16i = jnp.zeros((16,), jnp.int32)

        zero16 = jnp.zeros((16,), jnp.float32)
        one16 = jnp.ones((16,), jnp.float32)
        iota16 = lax.iota(jnp.int32, 16)
        zero16i = jnp.zeros((16,), jnp.int32)

        def zrow(r, carry):
            for k in range(F // 16):
                acc[r, pl.ds(k * 16, 16)] = zero16
            accc[r, :] = zero16
            return carry

        lax.fori_loop(0, RPT, zrow, 0)
        for k in range(2 * CH // 16):
            wl_eids[pl.ds(k * 16, 16)] = zero16i

        def drain(n_drain, nwl):
            for k in range(CH // 16):
                gbuf[pl.ds(k * 16, 16)] = wl_eids[pl.ds(k * 16, 16)]
            pltpu.async_copy(msg_hbm.at[gbuf], rows_v, sem).wait()

            def acc_group(g, carry):
                e0 = g * 16
                eidx = e0 + iota16
                emsk = eidx < n_drain
                rows16 = wl_rows[pl.ds(e0, 16)]
                for c in range(F):
                    csplat = jnp.full((16,), c, jnp.int32)
                    vals = plsc.load_gather(rows_v, [eidx, csplat], mask=emsk)
                    plsc.addupdate_scatter(acc, [rows16, csplat], vals, mask=emsk)
                plsc.addupdate_scatter(accc, [rows16, zero16i],
                                       jnp.where(emsk, 1.0, 0.0), mask=emsk)
                return carry

            lax.fori_loop(0, (n_drain + 15) // 16, acc_group, 0)
            for k in range(CH // 16):
                wl_rows[pl.ds(k * 16, 16)] = wl_rows[pl.ds(CH + k * 16, 16)]
                wl_eids[pl.ds(k * 16, 16)] = wl_eids[pl.ds(CH + k * 16, 16)]
            return nwl - n_drain

        def scan_subchunk(base, s, nwl):
            for g in range(CH // 16):
                off = s * CH + g * 16
                v = idx_v[pl.ds(off, 16)]
                local = v - lo
                msk = (local >= 0) & (local < RPT)
                eid = base + off + iota16
                plsc.store_compressed(wl_rows.at[pl.ds(nwl, 16)], local, mask=msk)
                plsc.store_compressed(wl_eids.at[pl.ds(nwl, 16)], eid, mask=msk)
                nwl = nwl + jnp.sum(jnp.where(msk, 1, 0))
            return lax.cond(nwl >= CH, lambda n: drain(CH, n), lambda n: n, nwl)

        def scan_block(b, nwl):
            base = b * SBLK
            pltpu.sync_copy(src_hbm.at[pl.ds(base, SBLK)], idx_v)

            def sub(s, nwl):
                return scan_subchunk(base, s, nwl)

            return lax.fori_loop(0, SBLK // CH, sub, nwl)

        nwl = lax.fori_loop(0, n_full, scan_block, 0)
        if tail:
            base = n_full * SBLK
            pltpu.sync_copy(src_hbm.at[pl.ds(base, tail)], idx_v.at[pl.ds(0, tail)])

            def sub_t(s, nwl):
                return scan_subchunk(base, s, nwl)

            nwl = lax.fori_loop(0, tail // CH, sub_t, nwl)
        drain(nwl, nwl)

        pltpu.sync_copy(acc, agg_hbm.at[pl.ds(lo, RPT)])
        pltpu.sync_copy(accc, cnt_hbm.at[pl.ds(lo, RPT)])

    return scatter_k(msg, src)


def _tc_msg_mlp(xg, ea, W1a, W1b, b1m, W2m, b2m):
    E, F = xg.shape
    H = W1a.shape[1]
    BE = 1600
    assert E % BE == 0

    def kern(xg_ref, ea_ref, w1a_ref, w1b_ref, b1_ref, w2_ref, b2_ref, out_ref):
        h = (jnp.dot(xg_ref[...], w1a_ref[...], preferred_element_type=jnp.float32)
             + jnp.dot(ea_ref[...], w1b_ref[...], preferred_element_type=jnp.float32)
             + b1_ref[...])
        h = jnp.maximum(h, 0.0)
        out_ref[...] = (jnp.dot(h, w2_ref[...], preferred_element_type=jnp.float32)
                        + b2_ref[...])

    return pl.pallas_call(
        kern,
        grid=(E // BE,),
        in_specs=[
            pl.BlockSpec((BE, F), lambda i: (i, 0)),
            pl.BlockSpec((BE, F), lambda i: (i, 0)),
            pl.BlockSpec((F, H), lambda i: (0, 0)),
            pl.BlockSpec((F, H), lambda i: (0, 0)),
            pl.BlockSpec((1, H), lambda i: (0, 0)),
            pl.BlockSpec((H, F), lambda i: (0, 0)),
            pl.BlockSpec((1, F), lambda i: (0, 0)),
        ],
        out_specs=pl.BlockSpec((BE, F), lambda i: (i, 0)),
        out_shape=jax.ShapeDtypeStruct((E, F), jnp.float32),
    )(xg, ea, W1a, W1b, b1m, W2m, b2m)


def _tc_update(x, agg_p, cnt_p, W1x, W1a, b1u, W2u, b2u, gamma, beta):
    N, F = x.shape

    def kern(x_ref, a_ref, c_ref, w1x_ref, w1a_ref, b1_ref, w2_ref, b2_ref,
             g_ref, be_ref, out_ref):
        cnt = jnp.maximum(c_ref[:N, :1], 1.0)
        agg = a_ref[:N] / cnt
        h = (jnp.dot(x_ref[...], w1x_ref[...], preferred_element_type=jnp.float32)
             + jnp.dot(agg, w1a_ref[...], preferred_element_type=jnp.float32)
             + b1_ref[...])
        h = jnp.maximum(h, 0.0)
        h = (jnp.dot(h, w2_ref[...], preferred_element_type=jnp.float32)
             + b2_ref[...])
        mu = jnp.mean(h, axis=0, keepdims=True)
        d = h - mu
        var = jnp.mean(d * d, axis=0, keepdims=True)
        out_ref[...] = d * lax.rsqrt(var + 1e-5) * g_ref[...] + be_ref[...]

    return pl.pallas_call(
        kern,
        out_shape=jax.ShapeDtypeStruct((N, F), jnp.float32),
    )(x, agg_p, cnt_p, W1x, W1a, b1u, W2u, b2u, gamma, beta)


def kernel(x, edge_index, edge_attr, W1m, b1m, W2m, b2m, W1u, b1u, W2u, b2u,
           gamma, beta):
    N, F = x.shape
    src = edge_index[0]
    tgt = edge_index[1]

    xg = _sc_gather(x, tgt)
    msg = _tc_msg_mlp(xg, edge_attr, W1m[:F], W1m[F:], b1m.reshape(1, -1),
                      W2m, b2m.reshape(1, -1))
    agg_p, cnt_p = _sc_scatter(msg, src, N)
    out = _tc_update(x, agg_p, cnt_p, W1u[:F], W1u[F:], b1u.reshape(1, -1),
                     W2u, b2u.reshape(1, -1), gamma.reshape(1, -1),
                     beta.reshape(1, -1))
    return out


# speedup vs baseline: 2.0467x; 2.0467x over previous
import functools

import jax
import jax.numpy as jnp
from jax import lax
from jax.experimental import pallas as pl
from jax.experimental.pallas import tpu as pltpu
from jax.experimental.pallas import tpu_sc as plsc

CH = 128
CW = 16


def _sc_gather(x, tgt):
    N, F = x.shape
    E = tgt.shape[0]
    info = plsc.get_sparse_core_info()
    NC, NS = info.num_cores, info.num_subcores
    NW = NC * NS
    assert E % CH == 0
    n_chunks = E // CH
    trips = (n_chunks + NW - 1) // NW

    mesh = plsc.VectorSubcoreMesh(core_axis_name="c", subcore_axis_name="s")

    @functools.partial(
        pl.kernel,
        out_type=jax.ShapeDtypeStruct((E, F), jnp.float32),
        mesh=mesh,
        scratch_types=[
            pltpu.VMEM((CH,), jnp.int32),
            pltpu.VMEM((CH, F), jnp.float32),
            pltpu.SemaphoreType.DMA,
        ],
    )
    def gather_k(x_hbm, tgt_hbm, out_hbm, idx_v, rows_v, sem):
        cid = lax.axis_index("c")
        sid = lax.axis_index("s")
        wid = sid * NC + cid

        def body(j, carry):
            chunk = wid + NW * j

            @pl.when(chunk < n_chunks)
            def _():
                base = chunk * CH
                pltpu.sync_copy(tgt_hbm.at[pl.ds(base, CH)], idx_v)
                pltpu.async_copy(x_hbm.at[idx_v], rows_v, sem).wait()
                pltpu.sync_copy(rows_v, out_hbm.at[pl.ds(base, CH)])

            return carry

        lax.fori_loop(0, trips, body, 0)

    return gather_k(x, tgt)


def _sc_scatter(msg, src, num_nodes):
    E, F = msg.shape
    info = plsc.get_sparse_core_info()
    NC, NS = info.num_cores, info.num_subcores
    NW = NC * NS
    NP = -(-num_nodes // (NW * 8)) * (NW * 8)
    RPT = NP // NW
    SBLK = 6400
    n_full = E // SBLK
    tail = E - n_full * SBLK
    assert tail % CH == 0 and n_full % 2 == 0

    mesh = plsc.VectorSubcoreMesh(core_axis_name="c", subcore_axis_name="s")

    @functools.partial(
        pl.kernel,
        out_type=(
            jax.ShapeDtypeStruct((NP * F,), jnp.float32),
            jax.ShapeDtypeStruct((NP * CW,), jnp.float32),
        ),
        mesh=mesh,
        scratch_types=[
            pltpu.VMEM((SBLK,), jnp.int32),
            pltpu.VMEM((SBLK,), jnp.int32),
            pltpu.VMEM((CH, F), jnp.float32),
            pltpu.VMEM((2 * CH,), jnp.int32),
            pltpu.VMEM((CH,), jnp.int32),
            pltpu.VMEM((RPT * F,), jnp.float32),
            pltpu.VMEM((RPT * CW,), jnp.float32),
            pltpu.SemaphoreType.DMA,
            pltpu.SemaphoreType.DMA,
            pltpu.SemaphoreType.DMA,
        ],
        compiler_params=pltpu.CompilerParams(needs_layout_passes=False),
    )
    def scatter_k(msg_hbm, src_hbm, agg_hbm, cnt_hbm,
                  idx_a, idx_b, rows_v, wl, gbuf, acc, accc,
                  sem, sa, sb):
        cid = lax.axis_index("c")
        sid = lax.axis_index("s")
        wid = sid * NC + cid
        lo = wid * RPT

        zero16 = jnp.zeros((16,), jnp.float32)
        one16 = jnp.ones((16,), jnp.float32)
        iota16 = lax.iota(jnp.int32, 16)
        zero16i = jnp.zeros((16,), jnp.int32)
        lane0 = iota16 == 0
        EMASK = (1 << 19) - 1

        def zfill(r, carry):
            acc[pl.ds(r * 16, 16)] = zero16
            return carry

        lax.fori_loop(0, RPT * F // 16, zfill, 0)

        def zfillc(r, carry):
            accc[pl.ds(r * 16, 16)] = zero16
            return carry

        lax.fori_loop(0, RPT * CW // 16, zfillc, 0)
        for k in range(2 * CH // 16):
            wl[pl.ds(k * 16, 16)] = zero16i

        def drain(n_drain, nwl):
            for k in range(CH // 16):
                gbuf[pl.ds(k * 16, 16)] = wl[pl.ds(k * 16, 16)] & EMASK
            pltpu.async_copy(msg_hbm.at[gbuf], rows_v, sem).wait()

            def acc_edge(e, carry):
                packed = plsc.load_gather(wl, [jnp.full((16,), 0, jnp.int32) + e])
                r = packed >> 19
                base = r * F + iota16
                for k in range(F // 16):
                    v = rows_v[e, pl.ds(k * 16, 16)]
                    plsc.addupdate_scatter(acc, [base + (k * 16)], v)
                plsc.addupdate_scatter(accc, [r * CW + iota16], one16,
                                       mask=lane0)
                return carry

            lax.fori_loop(0, n_drain, acc_edge, 0)
            for k in range(CH // 16):
                wl[pl.ds(k * 16, 16)] = wl[pl.ds(CH + k * 16, 16)]
            return nwl - n_drain

        def scan_subchunk(idx_v, base, s, nwl):
            for g in range(CH // 16):
                off = s * CH + g * 16
                v = idx_v[pl.ds(off, 16)]
                local = v - lo
                msk = (local >= 0) & (local < RPT)
                packed = (local << 19) | (base + off + iota16)
                plsc.store_compressed(wl.at[pl.ds(nwl, 16)], packed, mask=msk)
                nwl = nwl + jnp.sum(jnp.where(msk, 1, 0))
            return lax.cond(nwl >= CH, lambda n: drain(CH, n), lambda n: n, nwl)

        def scan_buf(idx_v, b, nwl):
            def sub(s, nwl):
                return scan_subchunk(idx_v, b * SBLK, s, nwl)

            return lax.fori_loop(0, SBLK // CH, sub, nwl)

        cp_a = pltpu.async_copy(src_hbm.at[pl.ds(0, SBLK)], idx_a, sa)
        cp_b = pltpu.async_copy(src_hbm.at[pl.ds(SBLK, SBLK)], idx_b, sb)

        def scan_pair(j, nwl):
            b0 = 2 * j
            pltpu.make_async_copy(src_hbm.at[pl.ds(0, SBLK)], idx_a, sa).wait()
            nwl = scan_buf(idx_a, b0, nwl)

            @pl.when(b0 + 2 < n_full)
            def _():
                pltpu.async_copy(
                    src_hbm.at[pl.ds((b0 + 2) * SBLK, SBLK)], idx_a, sa)

            pltpu.make_async_copy(src_hbm.at[pl.ds(0, SBLK)], idx_b, sb).wait()
            nwl = scan_buf(idx_b, b0 + 1, nwl)

            @pl.when(b0 + 3 < n_full)
            def _():
                pltpu.async_copy(
                    src_hbm.at[pl.ds((b0 + 3) * SBLK, SBLK)], idx_b, sb)

            return nwl

        nwl = lax.fori_loop(0, n_full // 2, scan_pair, 0)
        if tail:
            base = n_full * SBLK
            pltpu.sync_copy(src_hbm.at[pl.ds(base, tail)], idx_a.at[pl.ds(0, tail)])

            def sub_t(s, nwl):
                return scan_subchunk(idx_a, base, s, nwl)

            nwl = lax.fori_loop(0, tail // CH, sub_t, nwl)
        drain(nwl, nwl)

        pltpu.sync_copy(acc, agg_hbm.at[pl.ds(lo * F, RPT * F)])
        pltpu.sync_copy(accc, cnt_hbm.at[pl.ds(lo * CW, RPT * CW)])

    agg_f, cnt_f = scatter_k(msg, src)
    return agg_f.reshape(NP, F), cnt_f.reshape(NP, CW)


def _tc_msg_mlp(xg, ea, W1a, W1b, b1m, W2m, b2m):
    E, F = xg.shape
    H = W1a.shape[1]
    BE = 1600
    assert E % BE == 0

    def kern(xg_ref, ea_ref, w1a_ref, w1b_ref, b1_ref, w2_ref, b2_ref, out_ref):
        h = (jnp.dot(xg_ref[...], w1a_ref[...], preferred_element_type=jnp.float32)
             + jnp.dot(ea_ref[...], w1b_ref[...], preferred_element_type=jnp.float32)
             + b1_ref[...])
        h = jnp.maximum(h, 0.0)
        out_ref[...] = (jnp.dot(h, w2_ref[...], preferred_element_type=jnp.float32)
                        + b2_ref[...])

    return pl.pallas_call(
        kern,
        grid=(E // BE,),
        in_specs=[
            pl.BlockSpec((BE, F), lambda i: (i, 0)),
            pl.BlockSpec((BE, F), lambda i: (i, 0)),
            pl.BlockSpec((F, H), lambda i: (0, 0)),
            pl.BlockSpec((F, H), lambda i: (0, 0)),
            pl.BlockSpec((1, H), lambda i: (0, 0)),
            pl.BlockSpec((H, F), lambda i: (0, 0)),
            pl.BlockSpec((1, F), lambda i: (0, 0)),
        ],
        out_specs=pl.BlockSpec((BE, F), lambda i: (i, 0)),
        out_shape=jax.ShapeDtypeStruct((E, F), jnp.float32),
    )(xg, ea, W1a, W1b, b1m, W2m, b2m)


def _tc_update(x, agg_p, cnt_p, W1x, W1a, b1u, W2u, b2u, gamma, beta):
    N, F = x.shape

    def kern(x_ref, a_ref, c_ref, w1x_ref, w1a_ref, b1_ref, w2_ref, b2_ref,
             g_ref, be_ref, out_ref):
        cnt = jnp.maximum(c_ref[:N, :1], 1.0)
        agg = a_ref[:N] / cnt
        h = (jnp.dot(x_ref[...], w1x_ref[...], preferred_element_type=jnp.float32)
             + jnp.dot(agg, w1a_ref[...], preferred_element_type=jnp.float32)
             + b1_ref[...])
        h = jnp.maximum(h, 0.0)
        h = (jnp.dot(h, w2_ref[...], preferred_element_type=jnp.float32)
             + b2_ref[...])
        mu = jnp.mean(h, axis=0, keepdims=True)
        d = h - mu
        var = jnp.mean(d * d, axis=0, keepdims=True)
        out_ref[...] = d * lax.rsqrt(var + 1e-5) * g_ref[...] + be_ref[...]

    return pl.pallas_call(
        kern,
        out_shape=jax.ShapeDtypeStruct((N, F), jnp.float32),
    )(x, agg_p, cnt_p, W1x, W1a, b1u, W2u, b2u, gamma, beta)


def kernel(x, edge_index, edge_attr, W1m, b1m, W2m, b2m, W1u, b1u, W2u, b2u,
           gamma, beta):
    N, F = x.shape
    src = edge_index[0]
    tgt = edge_index[1]

    xg = _sc_gather(x, tgt)
    msg = _tc_msg_mlp(xg, edge_attr, W1m[:F], W1m[F:], b1m.reshape(1, -1),
                      W2m, b2m.reshape(1, -1))
    agg_p, cnt_p = _sc_scatter(msg, src, N)
    out = _tc_update(x, agg_p, cnt_p, W1u[:F], W1u[F:], b1u.reshape(1, -1),
                     W2u, b2u.reshape(1, -1), gamma.reshape(1, -1),
                     beta.reshape(1, -1))
    return out
